# baseline (speedup 1.0000x reference)
import functools

import jax
import jax.numpy as jnp
from jax import lax
from jax.experimental import pallas as pl
from jax.experimental.pallas import tpu as pltpu
from jax.experimental.pallas import tpu_sc as plsc

_NC = 2
_NS = 16
_NW = _NC * _NS
_LANES = 16
_BLK = 40
_NSLOT = 5


@functools.partial(jax.jit, static_argnames=("n", "d", "b_total"))
def _sc_gather(features, idx_flat, n, d, b_total):
    rows_per_w = b_total // _NW
    n_full = rows_per_w // _BLK
    rem = rows_per_w - n_full * _BLK
    idx_pad = -(-rows_per_w // _BLK) * _BLK
    rows_pad = -(-(n + 8) // (_NS * 8)) * (_NS * 8)
    stripe = (n // _NS) // 8 * 8

    mesh = plsc.VectorSubcoreMesh(
        core_axis_name="c", subcore_axis_name="s",
        num_cores=_NC, num_subcores=_NS)

    @functools.partial(
        pl.kernel,
        out_type=jax.ShapeDtypeStruct((b_total, d), jnp.float32),
        mesh=mesh,
        scratch_types=[
            pltpu.VMEM((idx_pad,), jnp.int32),
            pltpu.VMEM((_NSLOT, _BLK, d), jnp.float32),
            pltpu.VMEM_SHARED((rows_pad, d), jnp.float32),
            [pltpu.SemaphoreType.DMA] * _NSLOT,
            [pltpu.SemaphoreType.DMA] * _NSLOT,
            pltpu.SemaphoreType.DMA,
        ],
    )
    def body(feat_hbm, idx_hbm, out_hbm, idx_v, rows_v, shared, gsems,
             ssems, isem):
        sid = lax.axis_index("s")
        wid = sid * _NC + lax.axis_index("c")

        idx_cp = pltpu.async_copy(
            idx_hbm.at[pl.ds(wid * rows_per_w, rows_per_w)],
            idx_v.at[pl.ds(0, rows_per_w)], isem)

        stage_base = sid * stripe
        chunks = []
        off = 0
        while off < stripe:
            size = min(_BLK, stripe - off)
            chunks.append((off, size))
            off += size
        hin = {}
        for c in range(min(2, len(chunks))):
            coff, csize = chunks[c]
            hin[c] = pltpu.async_copy(
                feat_hbm.at[pl.ds(stage_base + coff, csize)],
                rows_v.at[c].at[pl.ds(0, csize)], gsems[c])

        idx_cp.wait()
        zi16 = jnp.zeros((_LANES,), jnp.int32)
        for i in range(rows_per_w // _LANES, idx_pad // _LANES):
            idx_v[pl.ds(i * _LANES, _LANES)] = zi16

        @pl.loop(0, idx_pad // _LANES, unroll=8)
        def _remap(i):
            sl = pl.ds(i * _LANES, _LANES)
            v = idx_v[sl]
            idx_v[sl] = jnp.where(v < 0, n, v)

        for c, (coff, csize) in enumerate(chunks):
            slot = c % 2
            hin[c].wait()
            hout = pltpu.async_copy(
                rows_v.at[slot].at[pl.ds(0, csize)],
                shared.at[pl.ds(stage_base + coff, csize)], ssems[slot])
            hout.wait()
            if c + 2 < len(chunks):
                noff, nsize = chunks[c + 2]
                hin[c + 2] = pltpu.async_copy(
                    feat_hbm.at[pl.ds(stage_base + noff, nsize)],
                    rows_v.at[slot].at[pl.ds(0, nsize)], gsems[slot])

        leftover = n - _NS * stripe
        assert leftover >= 0 and leftover % 8 == 0 and leftover <= _BLK
        zeros16 = jnp.zeros((_LANES,), jnp.float32)

        @pl.when(sid == 0)
        def _stage_tail():
            if leftover:
                pltpu.sync_copy(
                    feat_hbm.at[pl.ds(_NS * stripe, leftover)],
                    rows_v.at[0].at[pl.ds(0, leftover)])
                pltpu.sync_copy(
                    rows_v.at[0].at[pl.ds(0, leftover)],
                    shared.at[pl.ds(_NS * stripe, leftover)])
            for r in range(8):
                for i in range(d // _LANES):
                    rows_v[0, r, pl.ds(i * _LANES, _LANES)] = zeros16
            pltpu.sync_copy(rows_v.at[0].at[pl.ds(0, 8)],
                            shared.at[pl.ds(n, 8)])

        plsc.subcore_barrier()

        out_base = wid * rows_per_w

        def fire_gather(b, j):
            return pltpu.async_copy(
                shared.at[idx_v.at[pl.ds(b * _BLK, _BLK)]], rows_v.at[j],
                gsems[j])

        def fire_scatter(b, j):
            return pltpu.async_copy(
                rows_v.at[j], out_hbm.at[pl.ds(out_base + b * _BLK, _BLK)],
                ssems[j])

        def wait_gather(j):
            pltpu.make_async_copy(feat_hbm.at[pl.ds(0, _BLK)],
                                  rows_v.at[j], gsems[j]).wait()

        def wait_scatter(j):
            pltpu.make_async_copy(rows_v.at[j],
                                  out_hbm.at[pl.ds(0, _BLK)],
                                  ssems[j]).wait()

        assert n_full % _NSLOT == 0 and n_full >= _NSLOT
        nb_tot = n_full + (1 if rem else 0)

        for j in range(_NSLOT):
            fire_gather(j, j)

        @pl.loop(0, n_full // _NSLOT)
        def _group(p):
            b0 = p * _NSLOT
            for j in range(_NSLOT):
                wait_gather(j)
                fire_scatter(b0 + j, j)
            for j in range(_NSLOT):
                wait_scatter(j)
                nxt = b0 + _NSLOT + j

                @pl.when(nxt < nb_tot)
                def _fire_next():
                    fire_gather(nxt, j)

        if rem:
            wait_gather(0)
            pltpu.sync_copy(
                rows_v.at[0].at[pl.ds(0, rem)],
                out_hbm.at[pl.ds(out_base + n_full * _BLK, rem)])

    return body(features, idx_flat)


def kernel(features, nidx):
    n, d = features.shape
    nn, k = nidx.shape
    b_total = nn * k
    assert b_total % _NW == 0 and (b_total // _NW) % 8 == 0
    out_flat = _sc_gather(features, nidx.reshape(-1), n, d, b_total)
    return out_flat.reshape(nn, k, d)


# speedup vs baseline: 1.0186x; 1.0186x over previous
import functools

import jax
import jax.numpy as jnp
from jax import lax
from jax.experimental import pallas as pl
from jax.experimental.pallas import tpu as pltpu
from jax.experimental.pallas import tpu_sc as plsc

_NC = 2
_NS = 16
_NW = _NC * _NS
_LANES = 16
_BLK = 56
_NSLOT = 5


@functools.partial(jax.jit, static_argnames=("n", "d", "b_total"))
def _sc_gather(features, idx_flat, n, d, b_total):
    rows_per_w = b_total // _NW
    n_full = rows_per_w // _BLK
    rem = rows_per_w - n_full * _BLK
    assert rows_per_w % _LANES == 0
    idx_pad = rows_per_w
    rows_pad = n + 1
    stripe = (n // _NS) // 8 * 8

    mesh = plsc.VectorSubcoreMesh(
        core_axis_name="c", subcore_axis_name="s",
        num_cores=_NC, num_subcores=_NS)

    @functools.partial(
        pl.kernel,
        out_type=jax.ShapeDtypeStruct((b_total, d), jnp.float32),
        mesh=mesh,
        scratch_types=[
            pltpu.VMEM((idx_pad,), jnp.int32),
            pltpu.VMEM((_NSLOT, _BLK, d), jnp.float32),
            pltpu.VMEM_SHARED((rows_pad, d), jnp.float32),
            [pltpu.SemaphoreType.DMA] * _NSLOT,
            [pltpu.SemaphoreType.DMA] * _NSLOT,
            pltpu.SemaphoreType.DMA,
        ],
    )
    def body(feat_hbm, idx_hbm, out_hbm, idx_v, rows_v, shared, gsems,
             ssems, isem):
        sid = lax.axis_index("s")
        wid = sid * _NC + lax.axis_index("c")

        idx_cp = pltpu.async_copy(
            idx_hbm.at[pl.ds(wid * rows_per_w, rows_per_w)],
            idx_v.at[pl.ds(0, rows_per_w)], isem)

        stage_base = sid * stripe
        chunks = []
        off = 0
        while off < stripe:
            size = min(_BLK, stripe - off)
            chunks.append((off, size))
            off += size
        hin = {}
        for c in range(min(2, len(chunks))):
            coff, csize = chunks[c]
            hin[c] = pltpu.async_copy(
                feat_hbm.at[pl.ds(stage_base + coff, csize)],
                rows_v.at[c].at[pl.ds(0, csize)], gsems[c])

        idx_cp.wait()
        zi16 = jnp.zeros((_LANES,), jnp.int32)
        for i in range(rows_per_w // _LANES, idx_pad // _LANES):
            idx_v[pl.ds(i * _LANES, _LANES)] = zi16

        @pl.loop(0, idx_pad // _LANES, unroll=8)
        def _remap(i):
            sl = pl.ds(i * _LANES, _LANES)
            v = idx_v[sl]
            idx_v[sl] = jnp.where(v < 0, n, v)

        for c, (coff, csize) in enumerate(chunks):
            slot = c % 2
            hin[c].wait()
            hout = pltpu.async_copy(
                rows_v.at[slot].at[pl.ds(0, csize)],
                shared.at[pl.ds(stage_base + coff, csize)], ssems[slot])
            hout.wait()
            if c + 2 < len(chunks):
                noff, nsize = chunks[c + 2]
                hin[c + 2] = pltpu.async_copy(
                    feat_hbm.at[pl.ds(stage_base + noff, nsize)],
                    rows_v.at[slot].at[pl.ds(0, nsize)], gsems[slot])

        leftover = n - _NS * stripe
        assert leftover >= 0 and leftover % 8 == 0 and leftover <= _BLK
        zeros16 = jnp.zeros((_LANES,), jnp.float32)

        @pl.when(sid == 0)
        def _stage_tail():
            if leftover:
                pltpu.sync_copy(
                    feat_hbm.at[pl.ds(_NS * stripe, leftover)],
                    rows_v.at[0].at[pl.ds(0, leftover)])
                pltpu.sync_copy(
                    rows_v.at[0].at[pl.ds(0, leftover)],
                    shared.at[pl.ds(_NS * stripe, leftover)])
            for i in range(d // _LANES):
                rows_v[0, 0, pl.ds(i * _LANES, _LANES)] = zeros16
            pltpu.sync_copy(rows_v.at[0].at[pl.ds(0, 1)],
                            shared.at[pl.ds(n, 1)])

        plsc.subcore_barrier()

        out_base = wid * rows_per_w

        def fire_gather(b, j):
            return pltpu.async_copy(
                shared.at[idx_v.at[pl.ds(b * _BLK, _BLK)]], rows_v.at[j],
                gsems[j])

        def fire_scatter(b, j):
            return pltpu.async_copy(
                rows_v.at[j], out_hbm.at[pl.ds(out_base + b * _BLK, _BLK)],
                ssems[j])

        def wait_gather(j):
            pltpu.make_async_copy(feat_hbm.at[pl.ds(0, _BLK)],
                                  rows_v.at[j], gsems[j]).wait()

        def wait_scatter(j):
            pltpu.make_async_copy(rows_v.at[j],
                                  out_hbm.at[pl.ds(0, _BLK)],
                                  ssems[j]).wait()

        assert n_full >= _NSLOT
        nb_tot = n_full + (1 if rem else 0)
        steady = (n_full // _NSLOT) * _NSLOT

        for j in range(_NSLOT):
            fire_gather(j, j)

        @pl.loop(0, n_full // _NSLOT)
        def _group(p):
            b0 = p * _NSLOT
            for j in range(_NSLOT):
                wait_gather(j)
                fire_scatter(b0 + j, j)
            for j in range(_NSLOT):
                wait_scatter(j)
                nxt = b0 + _NSLOT + j

                @pl.when(nxt < n_full)
                def _fire_next():
                    fire_gather(nxt, j)

        for b in range(steady, n_full):
            wait_gather(b % _NSLOT)
            fire_scatter(b, b % _NSLOT)
        if rem:
            j = n_full % _NSLOT
            hrem = pltpu.async_copy(
                shared.at[idx_v.at[pl.ds(n_full * _BLK, rem)]],
                rows_v.at[j].at[pl.ds(0, rem)], gsems[j])
            hrem.wait()
            pltpu.sync_copy(
                rows_v.at[j].at[pl.ds(0, rem)],
                out_hbm.at[pl.ds(out_base + n_full * _BLK, rem)])
        for b in range(steady, n_full):
            wait_scatter(b % _NSLOT)

    return body(features, idx_flat)


def kernel(features, nidx):
    n, d = features.shape
    nn, k = nidx.shape
    b_total = nn * k
    assert b_total % _NW == 0 and (b_total // _NW) % 8 == 0
    out_flat = _sc_gather(features, nidx.reshape(-1), n, d, b_total)
    return out_flat.reshape(nn, k, d)
